# baseline (speedup 1.0000x reference)
import functools

import jax
import jax.numpy as jnp
from jax import lax
from jax.experimental import pallas as pl
from jax.experimental.pallas import tpu as pltpu
from jax.experimental.pallas import tpu_sc as plsc

N_PAD = 10240
D = 128
DEG_W = 16
CHUNK = 64
NC = 2
NS = 16
L = 16
ROWS_PER_TILE = N_PAD // NS
STAGE_STEPS = ROWS_PER_TILE // CHUNK
DROWS = N_PAD // D
DROWS_ALLOC = 128
DROWS_PER_TILE = DROWS_ALLOC // NS
SPLIT0 = 0.70


def _sc_body(feat, srcs, dsts, oh, sums, degs,
             acc, dacc,
             isA, idA, ihA, ilA, rwA, ohA, smA, smA2,
             isB, idB, ihB, ilB, rwB, ohB, smB, smB2,
             dstage,
             *, cpt_total, cpt0):
    c = lax.axis_index("c")
    s = lax.axis_index("s")
    r0 = s * ROWS_PER_TILE
    rows = rwA

    zero = jnp.zeros((L,), jnp.float32)

    def zbody(i, carry):
        for j in range(D // L):
            rows[i, pl.ds(j * L, L)] = zero
        return carry
    lax.fori_loop(0, CHUNK, zbody, 0)
    for k in range(STAGE_STEPS):
        pltpu.sync_copy(rows, acc.at[pl.ds(r0 + k * CHUNK, CHUNK)])
    pltpu.sync_copy(rows.at[pl.ds(0, DROWS_PER_TILE)],
                    dacc.at[pl.ds(s * DROWS_PER_TILE, DROWS_PER_TILE)])

    plsc.subcore_barrier()

    bufA = (isA, idA, ihA, ilA, rwA, ohA, smA)
    bufB = (isB, idB, ihB, ilB, rwB, ohB, smB)

    def pipeline(base_t, cpt):
        def load_start(ci, buf):
            ixs, ixd, ixh, ixl, rw, ohr, sm = buf
            b = base_t + ci * CHUNK
            pltpu.sync_copy(srcs.at[pl.ds(b, CHUNK)], ixs)
            pltpu.sync_copy(dsts.at[pl.ds(b, CHUNK)], ixd)
            for j in range(CHUNK // L):
                dv = ixd[pl.ds(j * L, L)]
                ixh[pl.ds(j * L, L)] = dv >> 7
                ixl[pl.ds(j * L, L)] = dv & 127
            pltpu.async_copy(feat.at[ixs], rw, sm)
            pltpu.async_copy(oh.at[ixl], ohr, sm)

        def wait_pair(buf):
            ixs, ixd, ixh, ixl, rw, ohr, sm = buf
            pltpu.make_async_copy(feat.at[ixs], rw, sm).wait()
            pltpu.make_async_copy(oh.at[ixl], ohr, sm).wait()

        def scat(buf, sm2):
            ixs, ixd, ixh, ixl, rw, ohr, sm = buf
            pltpu.async_copy(ohr, dacc.at[ixh], sm2, add=True)
            pltpu.sync_copy(rw, acc.at[ixd], add=True)
            pltpu.make_async_copy(ohr, dacc.at[ixh], sm2).wait()

        load_start(0, bufA)

        def step(i, carry):
            load_start(2 * i + 1, bufB)
            wait_pair(bufA)
            scat(bufA, smA2)
            load_start(2 * i + 2, bufA)
            wait_pair(bufB)
            scat(bufB, smB2)
            return carry
        lax.fori_loop(0, cpt // 2 - 1, step, 0)

        load_start(cpt - 1, bufB)
        wait_pair(bufA)
        scat(bufA, smA2)
        wait_pair(bufB)
        scat(bufB, smB2)

    @pl.when(c == 0)
    def _():
        pipeline(s * cpt_total * CHUNK, cpt0)

    @pl.when(c == 1)
    def _():
        pipeline((s * cpt_total + cpt0) * CHUNK, cpt_total - cpt0)

    plsc.subcore_barrier()

    for k in range(STAGE_STEPS):
        rr = r0 + k * CHUNK
        pltpu.sync_copy(acc.at[pl.ds(rr, CHUNK)], rows)
        pltpu.sync_copy(rows, sums.at[c, pl.ds(rr, CHUNK)])

    d0 = s * DROWS_PER_TILE
    pltpu.sync_copy(dacc.at[pl.ds(d0, DROWS_PER_TILE)], dstage)
    pltpu.sync_copy(dstage, degs.at[c, pl.ds(d0, DROWS_PER_TILE)])


def _combine_body(sums_ref, degs_ref, out_ref):
    ssum = sums_ref[0] + sums_ref[1]
    deg = degs_ref[0] + degs_ref[1]
    a_dim = ssum.shape[0]
    recip = jnp.where(deg > 0.0, 1.0 / jnp.maximum(deg, 1.0), 0.0)
    ii = lax.broadcasted_iota(jnp.int32, (a_dim, D, D), 1)
    jj = lax.broadcasted_iota(jnp.int32, (a_dim, D, D), 2)
    diag = jnp.where(ii == jj, recip[:, None, :], 0.0)
    out_ref[...] = lax.dot_general(
        diag, ssum, (((2,), (1,)), ((0,), (0,))),
        preferred_element_type=jnp.float32)


@jax.jit
def kernel(feat, edge_index):
    n, d = feat.shape
    e = edge_index.shape[1]
    step = NS * CHUNK * 4
    e_pad = ((e + step - 1) // step) * step
    pad = e_pad - e
    srcs = jnp.concatenate(
        [edge_index[0], jnp.zeros((pad,), jnp.int32)]) if pad else edge_index[0]
    dsts = jnp.concatenate(
        [edge_index[1], jnp.full((pad,), n, jnp.int32)]) if pad else edge_index[1]
    oh = jnp.eye(D, dtype=jnp.float32)

    cpt_total = e_pad // (NS * CHUNK)
    cpt0 = int(cpt_total * SPLIT0) & ~1

    mesh = plsc.VectorSubcoreMesh(core_axis_name="c", subcore_axis_name="s")
    sums, degs = pl.kernel(
        functools.partial(_sc_body, cpt_total=cpt_total, cpt0=cpt0),
        out_type=[
            jax.ShapeDtypeStruct((NC, N_PAD, D), jnp.float32),
            jax.ShapeDtypeStruct((NC, DROWS_ALLOC, D), jnp.float32),
        ],
        mesh=mesh,
        scratch_types=[
            pltpu.VMEM_SHARED((N_PAD, D), jnp.float32),
            pltpu.VMEM_SHARED((DROWS_ALLOC, D), jnp.float32),
        ] + 2 * [
            pltpu.VMEM((CHUNK,), jnp.int32),
            pltpu.VMEM((CHUNK,), jnp.int32),
            pltpu.VMEM((CHUNK,), jnp.int32),
            pltpu.VMEM((CHUNK,), jnp.int32),
            pltpu.VMEM((CHUNK, D), jnp.float32),
            pltpu.VMEM((CHUNK, D), jnp.float32),
            pltpu.SemaphoreType.DMA,
            pltpu.SemaphoreType.DMA,
        ] + [
            pltpu.VMEM((DROWS_PER_TILE, D), jnp.float32),
        ],
        name="sage_agg_sc",
    )(feat, srcs, dsts, oh)

    ab = 8
    grid = DROWS // ab
    sums3 = sums.reshape(NC, DROWS, D, D)
    out = pl.pallas_call(
        _combine_body,
        grid=(grid,),
        in_specs=[
            pl.BlockSpec((NC, ab, D, D), lambda i: (0, i, 0, 0)),
            pl.BlockSpec((NC, ab, D), lambda i: (0, i, 0)),
        ],
        out_specs=pl.BlockSpec((ab, D, D), lambda i: (i, 0, 0)),
        out_shape=jax.ShapeDtypeStruct((DROWS, D, D), jnp.float32),
    )(sums3, degs)
    return out.reshape(N_PAD, D)[:n]


# speedup vs baseline: 1.0312x; 1.0312x over previous
import functools

import jax
import jax.numpy as jnp
from jax import lax
from jax.experimental import pallas as pl
from jax.experimental.pallas import tpu as pltpu
from jax.experimental.pallas import tpu_sc as plsc

N_PAD = 10240
D = 128
DEG_W = 16
CHUNK = 64
NC = 2
NS = 16
L = 16
ROWS_PER_TILE = N_PAD // NS
STAGE_STEPS = ROWS_PER_TILE // CHUNK
DROWS = N_PAD // D
DROWS_ALLOC = 128
DROWS_PER_TILE = DROWS_ALLOC // NS
SPLIT0 = 0.67


def _sc_body(feat, srcs, dsts, oh, sums, degs,
             acc, dacc,
             isA, idA, ihA, ilA, rwA, ohA, smA, smA2,
             isB, idB, ihB, ilB, rwB, ohB, smB, smB2,
             dstage,
             *, cpt_total, cpt0):
    c = lax.axis_index("c")
    s = lax.axis_index("s")
    r0 = s * ROWS_PER_TILE
    rows = rwA

    zero = jnp.zeros((L,), jnp.float32)

    def zbody(i, carry):
        for j in range(D // L):
            rows[i, pl.ds(j * L, L)] = zero
        return carry
    lax.fori_loop(0, CHUNK, zbody, 0)
    for k in range(STAGE_STEPS):
        pltpu.sync_copy(rows, acc.at[pl.ds(r0 + k * CHUNK, CHUNK)])
    pltpu.sync_copy(rows.at[pl.ds(0, DROWS_PER_TILE)],
                    dacc.at[pl.ds(s * DROWS_PER_TILE, DROWS_PER_TILE)])

    plsc.subcore_barrier()

    bufA = (isA, idA, ihA, ilA, rwA, ohA, smA)
    bufB = (isB, idB, ihB, ilB, rwB, ohB, smB)

    def pipeline(base_t, cpt):
        def load_start(ci, buf):
            ixs, ixd, ixh, ixl, rw, ohr, sm = buf
            b = base_t + ci * CHUNK
            pltpu.sync_copy(srcs.at[pl.ds(b, CHUNK)], ixs)
            pltpu.sync_copy(dsts.at[pl.ds(b, CHUNK)], ixd)
            for j in range(CHUNK // L):
                dv = ixd[pl.ds(j * L, L)]
                ixh[pl.ds(j * L, L)] = dv >> 7
                ixl[pl.ds(j * L, L)] = dv & 127
            pltpu.async_copy(feat.at[ixs], rw, sm)
            pltpu.async_copy(oh.at[ixl], ohr, sm)

        def wait_pair(buf):
            ixs, ixd, ixh, ixl, rw, ohr, sm = buf
            pltpu.make_async_copy(feat.at[ixs], rw, sm).wait()
            pltpu.make_async_copy(oh.at[ixl], ohr, sm).wait()

        def scat(buf, sm2):
            ixs, ixd, ixh, ixl, rw, ohr, sm = buf
            pltpu.async_copy(ohr, dacc.at[ixh], sm2, add=True)
            pltpu.sync_copy(rw, acc.at[ixd], add=True)
            pltpu.make_async_copy(ohr, dacc.at[ixh], sm2).wait()

        load_start(0, bufA)

        def step(i, carry):
            load_start(2 * i + 1, bufB)
            wait_pair(bufA)
            scat(bufA, smA2)
            load_start(2 * i + 2, bufA)
            wait_pair(bufB)
            scat(bufB, smB2)
            return carry
        lax.fori_loop(0, cpt // 2 - 1, step, 0)

        load_start(cpt - 1, bufB)
        wait_pair(bufA)
        scat(bufA, smA2)
        wait_pair(bufB)
        scat(bufB, smB2)

    @pl.when(c == 0)
    def _():
        pipeline(s * cpt_total * CHUNK, cpt0)

    @pl.when(c == 1)
    def _():
        pipeline((s * cpt_total + cpt0) * CHUNK, cpt_total - cpt0)

    plsc.subcore_barrier()

    for k in range(STAGE_STEPS):
        rr = r0 + k * CHUNK
        pltpu.sync_copy(acc.at[pl.ds(rr, CHUNK)], rows)
        pltpu.sync_copy(rows, sums.at[c, pl.ds(rr, CHUNK)])

    d0 = s * DROWS_PER_TILE
    pltpu.sync_copy(dacc.at[pl.ds(d0, DROWS_PER_TILE)], dstage)
    pltpu.sync_copy(dstage, degs.at[c, pl.ds(d0, DROWS_PER_TILE)])


def _combine_body(sums_ref, degs_ref, out_ref):
    ssum = sums_ref[0] + sums_ref[1]
    deg = degs_ref[0] + degs_ref[1]
    a_dim = ssum.shape[0]
    recip = jnp.where(deg > 0.0, 1.0 / jnp.maximum(deg, 1.0), 0.0)
    ii = lax.broadcasted_iota(jnp.int32, (a_dim, D, D), 1)
    jj = lax.broadcasted_iota(jnp.int32, (a_dim, D, D), 2)
    diag = jnp.where(ii == jj, recip[:, None, :], 0.0)
    out_ref[...] = lax.dot_general(
        diag, ssum, (((2,), (1,)), ((0,), (0,))),
        preferred_element_type=jnp.float32)


@jax.jit
def kernel(feat, edge_index):
    n, d = feat.shape
    e = edge_index.shape[1]
    step = NS * CHUNK * 4
    e_pad = ((e + step - 1) // step) * step
    pad = e_pad - e
    srcs = jnp.concatenate(
        [edge_index[0], jnp.zeros((pad,), jnp.int32)]) if pad else edge_index[0]
    dsts = jnp.concatenate(
        [edge_index[1], jnp.full((pad,), n, jnp.int32)]) if pad else edge_index[1]
    oh = jnp.eye(D, dtype=jnp.float32)

    cpt_total = e_pad // (NS * CHUNK)
    cpt0 = int(cpt_total * SPLIT0) & ~1

    mesh = plsc.VectorSubcoreMesh(core_axis_name="c", subcore_axis_name="s")
    sums, degs = pl.kernel(
        functools.partial(_sc_body, cpt_total=cpt_total, cpt0=cpt0),
        out_type=[
            jax.ShapeDtypeStruct((NC, N_PAD, D), jnp.float32),
            jax.ShapeDtypeStruct((NC, DROWS_ALLOC, D), jnp.float32),
        ],
        mesh=mesh,
        scratch_types=[
            pltpu.VMEM_SHARED((N_PAD, D), jnp.float32),
            pltpu.VMEM_SHARED((DROWS_ALLOC, D), jnp.float32),
        ] + 2 * [
            pltpu.VMEM((CHUNK,), jnp.int32),
            pltpu.VMEM((CHUNK,), jnp.int32),
            pltpu.VMEM((CHUNK,), jnp.int32),
            pltpu.VMEM((CHUNK,), jnp.int32),
            pltpu.VMEM((CHUNK, D), jnp.float32),
            pltpu.VMEM((CHUNK, D), jnp.float32),
            pltpu.SemaphoreType.DMA,
            pltpu.SemaphoreType.DMA,
        ] + [
            pltpu.VMEM((DROWS_PER_TILE, D), jnp.float32),
        ],
        name="sage_agg_sc",
    )(feat, srcs, dsts, oh)

    ab = 8
    grid = DROWS // ab
    sums3 = sums.reshape(NC, DROWS, D, D)
    out = pl.pallas_call(
        _combine_body,
        grid=(grid,),
        in_specs=[
            pl.BlockSpec((NC, ab, D, D), lambda i: (0, i, 0, 0)),
            pl.BlockSpec((NC, ab, D), lambda i: (0, i, 0)),
        ],
        out_specs=pl.BlockSpec((ab, D, D), lambda i: (i, 0, 0)),
        out_shape=jax.ShapeDtypeStruct((DROWS, D, D), jnp.float32),
    )(sums3, degs)
    return out.reshape(N_PAD, D)[:n]


# speedup vs baseline: 1.0640x; 1.0319x over previous
import functools

import jax
import jax.numpy as jnp
from jax import lax
from jax.experimental import pallas as pl
from jax.experimental.pallas import tpu as pltpu
from jax.experimental.pallas import tpu_sc as plsc

N_PAD = 10240
D = 128
DEG_W = 16
CHUNK = 64
NC = 2
NS = 16
L = 16
ROWS_PER_TILE = N_PAD // NS
STAGE_STEPS = ROWS_PER_TILE // CHUNK
DROWS = N_PAD // D
DROWS_ALLOC = 128
DROWS_PER_TILE = DROWS_ALLOC // NS
SPLIT0 = 0.66


def _sc_body(feat, srcs, dsts, oh, sums, degs,
             acc, dacc,
             isA, idA, ihA, ilA, rwA, ohA, smA, smA2,
             isB, idB, ihB, ilB, rwB, ohB, smB, smB2,
             dstage,
             *, cpt_total, cpt0):
    c = lax.axis_index("c")
    s = lax.axis_index("s")
    r0 = s * ROWS_PER_TILE
    rows = rwA

    zero = jnp.zeros((L,), jnp.float32)

    def zbody(i, carry):
        for j in range(D // L):
            rows[i, pl.ds(j * L, L)] = zero
        return carry
    lax.fori_loop(0, CHUNK, zbody, 0)
    for k in range(STAGE_STEPS):
        pltpu.sync_copy(rows, acc.at[pl.ds(r0 + k * CHUNK, CHUNK)])
    pltpu.sync_copy(rows.at[pl.ds(0, DROWS_PER_TILE)],
                    dacc.at[pl.ds(s * DROWS_PER_TILE, DROWS_PER_TILE)])

    plsc.subcore_barrier()

    bufA = (isA, idA, ihA, ilA, rwA, ohA, smA)
    bufB = (isB, idB, ihB, ilB, rwB, ohB, smB)

    def pipeline(base_t, cpt):
        def load_start(ci, buf, sm2):
            ixs, ixd, ixh, ixl, rw, ohr, sm = buf
            b = base_t + ci * CHUNK
            pltpu.async_copy(srcs.at[pl.ds(b, CHUNK)], ixs, sm2)
            pltpu.async_copy(dsts.at[pl.ds(b, CHUNK)], ixd, sm2)
            pltpu.make_async_copy(srcs.at[pl.ds(b, CHUNK)], ixs, sm2).wait()
            pltpu.make_async_copy(dsts.at[pl.ds(b, CHUNK)], ixd, sm2).wait()
            for j in range(CHUNK // L):
                dv = ixd[pl.ds(j * L, L)]
                ixh[pl.ds(j * L, L)] = dv >> 7
                ixl[pl.ds(j * L, L)] = dv & 127
            pltpu.async_copy(feat.at[ixs], rw, sm)
            pltpu.async_copy(oh.at[ixl], ohr, sm)

        def wait_pair(buf):
            ixs, ixd, ixh, ixl, rw, ohr, sm = buf
            pltpu.make_async_copy(feat.at[ixs], rw, sm).wait()
            pltpu.make_async_copy(oh.at[ixl], ohr, sm).wait()

        def scat(buf, sm2):
            ixs, ixd, ixh, ixl, rw, ohr, sm = buf
            pltpu.async_copy(ohr, dacc.at[ixh], sm2, add=True)
            pltpu.sync_copy(rw, acc.at[ixd], add=True)
            pltpu.make_async_copy(ohr, dacc.at[ixh], sm2).wait()

        load_start(0, bufA, smA2)

        def step(i, carry):
            load_start(2 * i + 1, bufB, smB2)
            wait_pair(bufA)
            scat(bufA, smA2)
            load_start(2 * i + 2, bufA, smA2)
            wait_pair(bufB)
            scat(bufB, smB2)
            return carry
        lax.fori_loop(0, cpt // 2 - 1, step, 0)

        load_start(cpt - 1, bufB, smB2)
        wait_pair(bufA)
        scat(bufA, smA2)
        wait_pair(bufB)
        scat(bufB, smB2)

    @pl.when(c == 0)
    def _():
        pipeline(s * cpt_total * CHUNK, cpt0)

    @pl.when(c == 1)
    def _():
        pipeline((s * cpt_total + cpt0) * CHUNK, cpt_total - cpt0)

    plsc.subcore_barrier()

    for k in range(STAGE_STEPS):
        rr = r0 + k * CHUNK
        pltpu.sync_copy(acc.at[pl.ds(rr, CHUNK)], rows)
        pltpu.sync_copy(rows, sums.at[c, pl.ds(rr, CHUNK)])

    d0 = s * DROWS_PER_TILE
    pltpu.sync_copy(dacc.at[pl.ds(d0, DROWS_PER_TILE)], dstage)
    pltpu.sync_copy(dstage, degs.at[c, pl.ds(d0, DROWS_PER_TILE)])


def _combine_body(sums_ref, degs_ref, out_ref):
    ssum = sums_ref[0] + sums_ref[1]
    deg = degs_ref[0] + degs_ref[1]
    a_dim = ssum.shape[0]
    recip = jnp.where(deg > 0.0, 1.0 / jnp.maximum(deg, 1.0), 0.0)
    ii = lax.broadcasted_iota(jnp.int32, (a_dim, D, D), 1)
    jj = lax.broadcasted_iota(jnp.int32, (a_dim, D, D), 2)
    diag = jnp.where(ii == jj, recip[:, None, :], 0.0)
    out_ref[...] = lax.dot_general(
        diag, ssum, (((2,), (1,)), ((0,), (0,))),
        preferred_element_type=jnp.float32)


@jax.jit
def kernel(feat, edge_index):
    n, d = feat.shape
    e = edge_index.shape[1]
    step = NS * CHUNK * 4
    e_pad = ((e + step - 1) // step) * step
    pad = e_pad - e
    srcs = jnp.concatenate(
        [edge_index[0], jnp.zeros((pad,), jnp.int32)]) if pad else edge_index[0]
    dsts = jnp.concatenate(
        [edge_index[1], jnp.full((pad,), n, jnp.int32)]) if pad else edge_index[1]
    oh = jnp.eye(D, dtype=jnp.float32)

    cpt_total = e_pad // (NS * CHUNK)
    cpt0 = int(cpt_total * SPLIT0) & ~1

    mesh = plsc.VectorSubcoreMesh(core_axis_name="c", subcore_axis_name="s")
    sums, degs = pl.kernel(
        functools.partial(_sc_body, cpt_total=cpt_total, cpt0=cpt0),
        out_type=[
            jax.ShapeDtypeStruct((NC, N_PAD, D), jnp.float32),
            jax.ShapeDtypeStruct((NC, DROWS_ALLOC, D), jnp.float32),
        ],
        mesh=mesh,
        scratch_types=[
            pltpu.VMEM_SHARED((N_PAD, D), jnp.float32),
            pltpu.VMEM_SHARED((DROWS_ALLOC, D), jnp.float32),
        ] + 2 * [
            pltpu.VMEM((CHUNK,), jnp.int32),
            pltpu.VMEM((CHUNK,), jnp.int32),
            pltpu.VMEM((CHUNK,), jnp.int32),
            pltpu.VMEM((CHUNK,), jnp.int32),
            pltpu.VMEM((CHUNK, D), jnp.float32),
            pltpu.VMEM((CHUNK, D), jnp.float32),
            pltpu.SemaphoreType.DMA,
            pltpu.SemaphoreType.DMA,
        ] + [
            pltpu.VMEM((DROWS_PER_TILE, D), jnp.float32),
        ],
        name="sage_agg_sc",
    )(feat, srcs, dsts, oh)

    ab = 8
    grid = DROWS // ab
    sums3 = sums.reshape(NC, DROWS, D, D)
    out = pl.pallas_call(
        _combine_body,
        grid=(grid,),
        in_specs=[
            pl.BlockSpec((NC, ab, D, D), lambda i: (0, i, 0, 0)),
            pl.BlockSpec((NC, ab, D), lambda i: (0, i, 0)),
        ],
        out_specs=pl.BlockSpec((ab, D, D), lambda i: (i, 0, 0)),
        out_shape=jax.ShapeDtypeStruct((DROWS, D, D), jnp.float32),
    )(sums3, degs)
    return out.reshape(N_PAD, D)[:n]


# speedup vs baseline: 1.0690x; 1.0046x over previous
import functools

import jax
import jax.numpy as jnp
from jax import lax
from jax.experimental import pallas as pl
from jax.experimental.pallas import tpu as pltpu
from jax.experimental.pallas import tpu_sc as plsc

N_PAD = 10240
D = 128
DEG_W = 16
CHUNK = 64
NC = 2
NS = 16
L = 16
ROWS_PER_TILE = N_PAD // NS
STAGE_STEPS = ROWS_PER_TILE // CHUNK
DROWS = N_PAD // D
DROWS_ALLOC = 128
DROWS_PER_TILE = DROWS_ALLOC // NS
SPLIT0 = 0.66


def _sc_body(feat, srcs, dsts, oh, sums, degs,
             acc, dacc,
             isA, idA, ihA, ilA, rwA, ohA, smA, smA2,
             isB, idB, ihB, ilB, rwB, ohB, smB, smB2,
             dstage,
             *, cpt_total, cpt0):
    c = lax.axis_index("c")
    s = lax.axis_index("s")
    r0 = s * ROWS_PER_TILE
    rows = rwA

    zero = jnp.zeros((L,), jnp.float32)

    def zbody(i, carry):
        for j in range(D // L):
            rows[i, pl.ds(j * L, L)] = zero
        return carry
    lax.fori_loop(0, CHUNK, zbody, 0)
    for k in range(STAGE_STEPS):
        pltpu.async_copy(rows, acc.at[pl.ds(r0 + k * CHUNK, CHUNK)], smA)
    pltpu.async_copy(rows.at[pl.ds(0, DROWS_PER_TILE)],
                     dacc.at[pl.ds(s * DROWS_PER_TILE, DROWS_PER_TILE)], smA)
    for k in range(STAGE_STEPS):
        pltpu.make_async_copy(rows, acc.at[pl.ds(r0 + k * CHUNK, CHUNK)],
                              smA).wait()
    pltpu.make_async_copy(rows.at[pl.ds(0, DROWS_PER_TILE)],
                          dacc.at[pl.ds(s * DROWS_PER_TILE, DROWS_PER_TILE)],
                          smA).wait()

    plsc.subcore_barrier()

    bufA = (isA, idA, ihA, ilA, rwA, ohA, smA)
    bufB = (isB, idB, ihB, ilB, rwB, ohB, smB)

    def pipeline(base_t, cpt):
        def load_start(ci, buf, sm2):
            ixs, ixd, ixh, ixl, rw, ohr, sm = buf
            b = base_t + ci * CHUNK
            pltpu.async_copy(srcs.at[pl.ds(b, CHUNK)], ixs, sm2)
            pltpu.async_copy(dsts.at[pl.ds(b, CHUNK)], ixd, sm2)
            pltpu.make_async_copy(srcs.at[pl.ds(b, CHUNK)], ixs, sm2).wait()
            pltpu.make_async_copy(dsts.at[pl.ds(b, CHUNK)], ixd, sm2).wait()
            for j in range(CHUNK // L):
                dv = ixd[pl.ds(j * L, L)]
                ixh[pl.ds(j * L, L)] = dv >> 7
                ixl[pl.ds(j * L, L)] = dv & 127
            pltpu.async_copy(feat.at[ixs], rw, sm)
            pltpu.async_copy(oh.at[ixl], ohr, sm)

        def wait_pair(buf):
            ixs, ixd, ixh, ixl, rw, ohr, sm = buf
            pltpu.make_async_copy(feat.at[ixs], rw, sm).wait()
            pltpu.make_async_copy(oh.at[ixl], ohr, sm).wait()

        def scat(buf, sm2):
            ixs, ixd, ixh, ixl, rw, ohr, sm = buf
            pltpu.async_copy(ohr, dacc.at[ixh], sm2, add=True)
            pltpu.sync_copy(rw, acc.at[ixd], add=True)
            pltpu.make_async_copy(ohr, dacc.at[ixh], sm2).wait()

        load_start(0, bufA, smA2)

        def step(i, carry):
            load_start(2 * i + 1, bufB, smB2)
            wait_pair(bufA)
            scat(bufA, smA2)
            load_start(2 * i + 2, bufA, smA2)
            wait_pair(bufB)
            scat(bufB, smB2)
            return carry
        lax.fori_loop(0, cpt // 2 - 1, step, 0)

        load_start(cpt - 1, bufB, smB2)
        wait_pair(bufA)
        scat(bufA, smA2)
        wait_pair(bufB)
        scat(bufB, smB2)

    @pl.when(c == 0)
    def _():
        pipeline(s * cpt_total * CHUNK, cpt0)

    @pl.when(c == 1)
    def _():
        pipeline((s * cpt_total + cpt0) * CHUNK, cpt_total - cpt0)

    plsc.subcore_barrier()

    def sl(k):
        return pl.ds(r0 + k * CHUNK, CHUNK)

    dbufs = (rwA, rwB)
    sin = (smA, smB)
    sout = (smA2, smB2)
    pltpu.async_copy(acc.at[sl(0)], dbufs[0], sin[0])
    for k in range(STAGE_STEPS):
        x = k % 2
        y = 1 - x
        if k + 1 < STAGE_STEPS:
            if k >= 1:
                pltpu.make_async_copy(dbufs[y], sums.at[c, sl(k - 1)],
                                      sout[y]).wait()
            pltpu.async_copy(acc.at[sl(k + 1)], dbufs[y], sin[y])
        pltpu.make_async_copy(acc.at[sl(k)], dbufs[x], sin[x]).wait()
        pltpu.async_copy(dbufs[x], sums.at[c, sl(k)], sout[x])
    lastx = (STAGE_STEPS - 1) % 2
    pltpu.make_async_copy(dbufs[1 - lastx], sums.at[c, sl(STAGE_STEPS - 2)],
                          sout[1 - lastx]).wait()
    pltpu.make_async_copy(dbufs[lastx], sums.at[c, sl(STAGE_STEPS - 1)],
                          sout[lastx]).wait()

    d0 = s * DROWS_PER_TILE
    pltpu.sync_copy(dacc.at[pl.ds(d0, DROWS_PER_TILE)], dstage)
    pltpu.sync_copy(dstage, degs.at[c, pl.ds(d0, DROWS_PER_TILE)])


def _combine_body(sums_ref, degs_ref, out_ref):
    ssum = sums_ref[0] + sums_ref[1]
    deg = degs_ref[0] + degs_ref[1]
    a_dim = ssum.shape[0]
    recip = jnp.where(deg > 0.0, 1.0 / jnp.maximum(deg, 1.0), 0.0)
    ii = lax.broadcasted_iota(jnp.int32, (a_dim, D, D), 1)
    jj = lax.broadcasted_iota(jnp.int32, (a_dim, D, D), 2)
    diag = jnp.where(ii == jj, recip[:, None, :], 0.0)
    out_ref[...] = lax.dot_general(
        diag, ssum, (((2,), (1,)), ((0,), (0,))),
        preferred_element_type=jnp.float32)


@jax.jit
def kernel(feat, edge_index):
    n, d = feat.shape
    e = edge_index.shape[1]
    step = NS * CHUNK * 4
    e_pad = ((e + step - 1) // step) * step
    pad = e_pad - e
    srcs = jnp.concatenate(
        [edge_index[0], jnp.zeros((pad,), jnp.int32)]) if pad else edge_index[0]
    dsts = jnp.concatenate(
        [edge_index[1], jnp.full((pad,), n, jnp.int32)]) if pad else edge_index[1]
    oh = jnp.eye(D, dtype=jnp.float32)

    cpt_total = e_pad // (NS * CHUNK)
    cpt0 = int(cpt_total * SPLIT0) & ~1

    mesh = plsc.VectorSubcoreMesh(core_axis_name="c", subcore_axis_name="s")
    sums, degs = pl.kernel(
        functools.partial(_sc_body, cpt_total=cpt_total, cpt0=cpt0),
        out_type=[
            jax.ShapeDtypeStruct((NC, N_PAD, D), jnp.float32),
            jax.ShapeDtypeStruct((NC, DROWS_ALLOC, D), jnp.float32),
        ],
        mesh=mesh,
        scratch_types=[
            pltpu.VMEM_SHARED((N_PAD, D), jnp.float32),
            pltpu.VMEM_SHARED((DROWS_ALLOC, D), jnp.float32),
        ] + 2 * [
            pltpu.VMEM((CHUNK,), jnp.int32),
            pltpu.VMEM((CHUNK,), jnp.int32),
            pltpu.VMEM((CHUNK,), jnp.int32),
            pltpu.VMEM((CHUNK,), jnp.int32),
            pltpu.VMEM((CHUNK, D), jnp.float32),
            pltpu.VMEM((CHUNK, D), jnp.float32),
            pltpu.SemaphoreType.DMA,
            pltpu.SemaphoreType.DMA,
        ] + [
            pltpu.VMEM((DROWS_PER_TILE, D), jnp.float32),
        ],
        name="sage_agg_sc",
    )(feat, srcs, dsts, oh)

    ab = 8
    grid = DROWS // ab
    sums3 = sums.reshape(NC, DROWS, D, D)
    out = pl.pallas_call(
        _combine_body,
        grid=(grid,),
        in_specs=[
            pl.BlockSpec((NC, ab, D, D), lambda i: (0, i, 0, 0)),
            pl.BlockSpec((NC, ab, D), lambda i: (0, i, 0)),
        ],
        out_specs=pl.BlockSpec((ab, D, D), lambda i: (i, 0, 0)),
        out_shape=jax.ShapeDtypeStruct((DROWS, D, D), jnp.float32),
    )(sums3, degs)
    return out.reshape(N_PAD, D)[:n]
